# baseline (speedup 1.0000x reference)
import functools

import jax
import jax.numpy as jnp
from jax import lax
from jax.experimental import pallas as pl
from jax.experimental.pallas import tpu as pltpu
from jax.experimental.pallas import tpu_sc as plsc

N = 10000
D = 128
DEG = 32

NC = 2
NS = 16
NW = NC * NS
NPW = 320
NP = NW * NPW
C = 4
E = C * DEG
NPH = NP // 2
NPWH = NPH // NW
NCHUNK = NPWH // C
NBUF = 2
RING = 4
NV = D // 16
TROWS = 624


def _sc_aggregate(inp, nbr_chunks, asc, bsc, node_base):
    mesh = plsc.VectorSubcoreMesh(
        core_axis_name="c", subcore_axis_name="s", num_cores=NC, num_subcores=NS)

    @functools.partial(
        pl.kernel,
        out_type=jax.ShapeDtypeStruct((NPH, D), jnp.float32),
        mesh=mesh,
        compiler_params=pltpu.CompilerParams(needs_layout_passes=False),
        scratch_types=[
            pltpu.VMEM((NP,), jnp.float32),
            pltpu.VMEM((NPWH + 16,), jnp.float32),
            pltpu.VMEM((RING, E), jnp.int32),
            pltpu.VMEM((NBUF, E, D), jnp.float32),
            pltpu.VMEM((RING, C, D), jnp.float32),
            pltpu.VMEM((E,), jnp.float32),
            pltpu.VMEM_SHARED((N, D), jnp.float32),
            pltpu.SemaphoreType.DMA((RING,)),
            pltpu.SemaphoreType.DMA((NBUF,)),
            pltpu.SemaphoreType.DMA((RING,)),
            pltpu.SemaphoreType.DMA,
        ],
    )
    def k(inp_hbm, nbr_hbm, asc_hbm, bsc_hbm, out_hbm,
          bsc_v, asc_v, idx_v, rows_v, acc_v, mask_v, tab_s,
          isem, gsem, osem, tsem):
        sid = lax.axis_index("s")
        wid = sid * NC + lax.axis_index("c")
        nbase = node_base + wid * NPWH
        obase = wid * NPWH
        tb = sid * TROWS
        pltpu.async_copy(inp_hbm.at[pl.ds(tb, TROWS)],
                         tab_s.at[pl.ds(tb, TROWS)], tsem)

        @pl.when(sid == 0)
        def _():
            pltpu.sync_copy(inp_hbm.at[pl.ds(NS * TROWS, N - NS * TROWS)],
                            tab_s.at[pl.ds(NS * TROWS, N - NS * TROWS)])

        for g in range(RING):
            pltpu.async_copy(nbr_hbm.at[wid, g], idx_v.at[g], isem.at[g])
        pltpu.sync_copy(bsc_hbm, bsc_v)
        pltpu.sync_copy(asc_hbm.at[pl.ds(nbase, NPWH)], asc_v.at[pl.ds(0, NPWH)])
        pltpu.make_async_copy(inp_hbm.at[pl.ds(tb, TROWS)],
                              tab_s.at[pl.ds(tb, TROWS)], tsem).wait()
        plsc.subcore_barrier()

        for g in range(NBUF):
            pltpu.make_async_copy(nbr_hbm.at[wid, g], idx_v.at[g],
                                  isem.at[g]).wait()
            pltpu.async_copy(tab_s.at[idx_v.at[g]], rows_v.at[g], gsem.at[g])

        @pl.loop(0, NCHUNK, step=RING)
        def _outer(g0):
            for q in range(RING):
                gc = g0 + q
                b = q % NBUF
                pltpu.make_async_copy(
                    tab_s.at[idx_v.at[q]], rows_v.at[b], gsem.at[b]).wait()
                av = asc_v[pl.ds(gc * C, 16)]
                for v in range(E // 16):
                    idx16 = idx_v[q, pl.ds(v * 16, 16)]
                    bs = plsc.load_gather(bsc_v, [idx16])
                    x = bs + lax.broadcast(av[v // 2], (16,))
                    z = jnp.exp(-jnp.abs(x))
                    mask_v[pl.ds(v * 16, 16)] = jnp.where(
                        x >= 0, 1.0 / (1.0 + z), z / (1.0 + z))
                @pl.when(gc + RING < NCHUNK)
                def _():
                    pltpu.async_copy(nbr_hbm.at[wid, gc + RING], idx_v.at[q],
                                     isem.at[q])
                @pl.when(gc >= RING)
                def _():
                    pltpu.make_async_copy(
                        acc_v.at[q],
                        out_hbm.at[pl.ds(obase + (gc - RING) * C, C)],
                        osem.at[q]).wait()

                @pl.loop(0, C)
                def _node(n):
                    acc = [jnp.zeros((16,), jnp.float32) for _ in range(NV)]
                    for h in range(DEG // 16):
                        mv = mask_v[pl.ds(n * DEG + h * 16, 16)]
                        for j in range(16):
                            e = n * DEG + h * 16 + j
                            m = lax.broadcast(mv[j], (16,))
                            for v in range(NV):
                                acc[v] = acc[v] + m * rows_v[b, e, pl.ds(v * 16, 16)]
                    for v in range(NV):
                        acc_v[q, n, pl.ds(v * 16, 16)] = acc[v]

                pltpu.async_copy(acc_v.at[q],
                                 out_hbm.at[pl.ds(obase + gc * C, C)], osem.at[q])
                @pl.when(gc + NBUF < NCHUNK)
                def _():
                    q2 = (q + NBUF) % RING
                    pltpu.make_async_copy(nbr_hbm.at[wid, gc + NBUF],
                                          idx_v.at[q2], isem.at[q2]).wait()
                    pltpu.async_copy(tab_s.at[idx_v.at[q2]], rows_v.at[b],
                                     gsem.at[b])

        for q in range(RING):
            pltpu.make_async_copy(
                acc_v.at[q],
                out_hbm.at[pl.ds(obase + (NCHUNK - RING + q) * C, C)],
                osem.at[q]).wait()

    return k(inp, nbr_chunks, asc, bsc)


def _scores_matmul(inp, wm_pad):
    blk = 1000

    def body(x_ref, w_ref, o_ref):
        o_ref[...] = jnp.dot(x_ref[...], w_ref[...],
                             preferred_element_type=jnp.float32)

    return pl.pallas_call(
        body,
        grid=(N // blk,),
        in_specs=[
            pl.BlockSpec((blk, D), lambda i: (i, 0)),
            pl.BlockSpec((D, 8), lambda i: (0, 0)),
        ],
        out_specs=pl.BlockSpec((blk, 8), lambda i: (i, 0)),
        out_shape=jax.ShapeDtypeStruct((N, 8), jnp.float32),
    )(inp, wm_pad)


def _support_matmul(x, agg, w, blk):
    rows = x.shape[0]

    def body(x_ref, g_ref, w_ref, o_ref):
        o_ref[...] = jnp.dot(x_ref[...] + g_ref[...], w_ref[...],
                             preferred_element_type=jnp.float32)

    return pl.pallas_call(
        body,
        grid=(rows // blk,),
        in_specs=[
            pl.BlockSpec((blk, D), lambda i: (i, 0)),
            pl.BlockSpec((blk, D), lambda i: (i, 0)),
            pl.BlockSpec((D, D), lambda i: (0, 0)),
        ],
        out_specs=pl.BlockSpec((blk, D), lambda i: (i, 0)),
        out_shape=jax.ShapeDtypeStruct((rows, D), jnp.float32),
    )(x, agg, w)


BM = 400
BK = 256
KJ1 = NPH // BK
KJ2 = (N - NPH - 16) // BK


def _adj_matmul_a(adj, sup_a, bias_row):
    def body(a_ref, s_ref, b_ref, o_ref):
        acc = jnp.dot(a_ref[...], s_ref[...], preferred_element_type=jnp.float32)

        @pl.when(pl.program_id(1) == 0)
        def _():
            o_ref[...] = acc + b_ref[...]

        @pl.when(pl.program_id(1) != 0)
        def _():
            o_ref[...] += acc

    return pl.pallas_call(
        body,
        grid=(N // BM, KJ1),
        in_specs=[
            pl.BlockSpec((BM, BK), lambda i, j: (i, j)),
            pl.BlockSpec((BK, D), lambda i, j: (j, 0)),
            pl.BlockSpec((1, D), lambda i, j: (0, 0)),
        ],
        out_specs=pl.BlockSpec((BM, D), lambda i, j: (i, 0)),
        out_shape=jax.ShapeDtypeStruct((N, D), jnp.float32),
    )(adj, sup_a, bias_row)


def _adj_matmul_b(p1, adj, sup_b, adj_tail, sup_tail):
    def body(p_ref, a_ref, s_ref, at_ref, st_ref, o_ref):
        acc = jnp.dot(a_ref[...], s_ref[...], preferred_element_type=jnp.float32)

        @pl.when(pl.program_id(1) == 0)
        def _():
            o_ref[...] = p_ref[...] + acc + jnp.dot(
                at_ref[...], st_ref[...], preferred_element_type=jnp.float32)

        @pl.when(pl.program_id(1) != 0)
        def _():
            o_ref[...] += acc

    return pl.pallas_call(
        body,
        grid=(N // BM, KJ2),
        in_specs=[
            pl.BlockSpec((BM, D), lambda i, j: (i, 0)),
            pl.BlockSpec((BM, BK), lambda i, j: (i, j + KJ1)),
            pl.BlockSpec((BK, D), lambda i, j: (j, 0)),
            pl.BlockSpec((BM, 16), lambda i, j: (i, 0)),
            pl.BlockSpec((16, D), lambda i, j: (0, 0)),
        ],
        out_specs=pl.BlockSpec((BM, D), lambda i, j: (i, 0)),
        out_shape=jax.ShapeDtypeStruct((N, D), jnp.float32),
    )(p1, adj, sup_b, adj_tail, sup_tail)


def kernel(input, adj, nbr_idx, weight_0, weights_mask0, bias):
    inp = input.astype(jnp.float32)
    nbr = nbr_idx.astype(jnp.int32)

    nbr_chunks = (jnp.zeros((NP, DEG), jnp.int32).at[:N].set(nbr)
                  .reshape(2, NW, NCHUNK, E))
    wm = weights_mask0.astype(jnp.float32).reshape(2 * D)
    wm_pad = jnp.zeros((D, 8), jnp.float32).at[:, 0].set(wm[:D]).at[:, 1].set(wm[D:])

    scores = _scores_matmul(inp, wm_pad)
    pad8 = jnp.zeros((NP - N, 8), jnp.float32)
    scores_pad = jnp.concatenate([scores, pad8], axis=0)
    asc = scores_pad[:, 0]
    bsc = scores_pad[:, 1]

    w0 = weight_0.astype(jnp.float32)
    adj32 = adj.astype(jnp.float32)
    bias_row = bias.astype(jnp.float32).reshape(1, D)
    adj_tail = jnp.zeros((N, 16), jnp.float32)

    agg_a = _sc_aggregate(inp, nbr_chunks[0], asc, bsc, 0)
    agg_b = _sc_aggregate(inp, nbr_chunks[1], asc, bsc, NPH)
    sup_a = _support_matmul(inp[:NPH], agg_a, w0, 1024)
    p1 = _adj_matmul_a(adj32, sup_a, bias_row)
    sup_b = _support_matmul(inp[NPH:N], agg_b[:N - NPH], w0, 976)
    return _adj_matmul_b(p1, adj32, sup_b[:KJ2 * BK], adj_tail,
                         sup_b[KJ2 * BK:])


# speedup vs baseline: 2.8570x; 2.8570x over previous
import functools

import jax
import jax.numpy as jnp
from jax import lax
from jax.experimental import pallas as pl
from jax.experimental.pallas import tpu as pltpu
from jax.experimental.pallas import tpu_sc as plsc

N = 10000
D = 128
DEG = 32

NC = 2
NS = 16
NW = NC * NS
NPW = 320
NP = NW * NPW
C = 4
E = C * DEG
NPH = NP // 2
NPWH = NPH // NW
NCHUNK = NPWH // C
NBUF = 2
RING = 4
NV = D // 16
TROWS = 624


def _sc_aggregate(inp, nbr_chunks, asc, bsc, node_base):
    mesh = plsc.VectorSubcoreMesh(
        core_axis_name="c", subcore_axis_name="s", num_cores=NC, num_subcores=NS)

    @functools.partial(
        pl.kernel,
        out_type=jax.ShapeDtypeStruct((NPH, D), jnp.float32),
        mesh=mesh,
        compiler_params=pltpu.CompilerParams(needs_layout_passes=False),
        scratch_types=[
            pltpu.VMEM((NP,), jnp.float32),
            pltpu.VMEM((NPWH + 16,), jnp.float32),
            pltpu.VMEM((RING, E), jnp.int32),
            pltpu.VMEM((NBUF, E, D), jnp.float32),
            pltpu.VMEM((RING, C, D), jnp.float32),
            pltpu.VMEM((E,), jnp.float32),
            pltpu.VMEM_SHARED((N, D), jnp.float32),
            pltpu.SemaphoreType.DMA((RING,)),
            pltpu.SemaphoreType.DMA((NBUF,)),
            pltpu.SemaphoreType.DMA((RING,)),
            pltpu.SemaphoreType.DMA,
        ],
    )
    def k(inp_hbm, nbr_hbm, asc_hbm, bsc_hbm, out_hbm,
          bsc_v, asc_v, idx_v, rows_v, acc_v, mask_v, tab_s,
          isem, gsem, osem, tsem):
        sid = lax.axis_index("s")
        wid = sid * NC + lax.axis_index("c")
        nbase = node_base + wid * NPWH
        obase = wid * NPWH
        tb = sid * TROWS
        pltpu.async_copy(inp_hbm.at[pl.ds(tb, TROWS)],
                         tab_s.at[pl.ds(tb, TROWS)], tsem)

        @pl.when(sid == 0)
        def _():
            pltpu.sync_copy(inp_hbm.at[pl.ds(NS * TROWS, N - NS * TROWS)],
                            tab_s.at[pl.ds(NS * TROWS, N - NS * TROWS)])

        for g in range(RING):
            pltpu.async_copy(nbr_hbm.at[wid, g], idx_v.at[g], isem.at[g])
        pltpu.sync_copy(bsc_hbm, bsc_v)
        pltpu.sync_copy(asc_hbm.at[pl.ds(nbase, NPWH)], asc_v.at[pl.ds(0, NPWH)])
        pltpu.make_async_copy(inp_hbm.at[pl.ds(tb, TROWS)],
                              tab_s.at[pl.ds(tb, TROWS)], tsem).wait()
        plsc.subcore_barrier()

        for g in range(NBUF):
            pltpu.make_async_copy(nbr_hbm.at[wid, g], idx_v.at[g],
                                  isem.at[g]).wait()
            pltpu.async_copy(tab_s.at[idx_v.at[g]], rows_v.at[g], gsem.at[g])

        @pl.loop(0, NCHUNK, step=RING)
        def _outer(g0):
            for q in range(RING):
                gc = g0 + q
                b = q % NBUF
                pltpu.make_async_copy(
                    tab_s.at[idx_v.at[q]], rows_v.at[b], gsem.at[b]).wait()
                av = asc_v[pl.ds(gc * C, 16)]
                for v in range(E // 16):
                    idx16 = idx_v[q, pl.ds(v * 16, 16)]
                    bs = plsc.load_gather(bsc_v, [idx16])
                    x = bs + lax.broadcast(av[v // 2], (16,))
                    z = jnp.exp(-jnp.abs(x))
                    mask_v[pl.ds(v * 16, 16)] = jnp.where(
                        x >= 0, 1.0 / (1.0 + z), z / (1.0 + z))
                @pl.when(gc + RING < NCHUNK)
                def _():
                    pltpu.async_copy(nbr_hbm.at[wid, gc + RING], idx_v.at[q],
                                     isem.at[q])
                @pl.when(gc >= RING)
                def _():
                    pltpu.make_async_copy(
                        acc_v.at[q],
                        out_hbm.at[pl.ds(obase + (gc - RING) * C, C)],
                        osem.at[q]).wait()

                @pl.loop(0, C)
                def _node(n):
                    acc = [jnp.zeros((16,), jnp.float32) for _ in range(NV)]
                    for h in range(DEG // 16):
                        mv = mask_v[pl.ds(n * DEG + h * 16, 16)]
                        for j in range(16):
                            e = n * DEG + h * 16 + j
                            m = lax.broadcast(mv[j], (16,))
                            for v in range(NV):
                                acc[v] = acc[v] + m * rows_v[b, e, pl.ds(v * 16, 16)]
                    for v in range(NV):
                        acc_v[q, n, pl.ds(v * 16, 16)] = acc[v]

                pltpu.async_copy(acc_v.at[q],
                                 out_hbm.at[pl.ds(obase + gc * C, C)], osem.at[q])
                @pl.when(gc + NBUF < NCHUNK)
                def _():
                    q2 = (q + NBUF) % RING
                    pltpu.make_async_copy(nbr_hbm.at[wid, gc + NBUF],
                                          idx_v.at[q2], isem.at[q2]).wait()
                    pltpu.async_copy(tab_s.at[idx_v.at[q2]], rows_v.at[b],
                                     gsem.at[b])

        for q in range(RING):
            pltpu.make_async_copy(
                acc_v.at[q],
                out_hbm.at[pl.ds(obase + (NCHUNK - RING + q) * C, C)],
                osem.at[q]).wait()

    return k(inp, nbr_chunks, asc, bsc)


def _scores_matmul(inp, wm_pad):
    blk = 1000

    def body(x_ref, w_ref, o_ref):
        o_ref[...] = jnp.dot(x_ref[...], w_ref[...],
                             preferred_element_type=jnp.float32)

    return pl.pallas_call(
        body,
        grid=(N // blk,),
        in_specs=[
            pl.BlockSpec((blk, D), lambda i: (i, 0)),
            pl.BlockSpec((D, 8), lambda i: (0, 0)),
        ],
        out_specs=pl.BlockSpec((blk, 8), lambda i: (i, 0)),
        out_shape=jax.ShapeDtypeStruct((N, 8), jnp.float32),
    )(inp, wm_pad)


def _support_matmul(x, agg, w, blk):
    rows = x.shape[0]

    def body(x_ref, g_ref, w_ref, o_ref):
        o_ref[...] = jnp.dot(x_ref[...] + g_ref[...], w_ref[...],
                             preferred_element_type=jnp.float32)

    return pl.pallas_call(
        body,
        grid=(rows // blk,),
        in_specs=[
            pl.BlockSpec((blk, D), lambda i: (i, 0)),
            pl.BlockSpec((blk, D), lambda i: (i, 0)),
            pl.BlockSpec((D, D), lambda i: (0, 0)),
        ],
        out_specs=pl.BlockSpec((blk, D), lambda i: (i, 0)),
        out_shape=jax.ShapeDtypeStruct((rows, D), jnp.float32),
    )(x, agg, w)


BM = 400
NVALID_B = N - NPH


def _adj_matmul_a(adj, sup_a, bias_row):
    def body(a_ref, s_ref, b_ref, o_ref):
        o_ref[...] = jnp.dot(a_ref[...], s_ref[...],
                             preferred_element_type=jnp.float32) + b_ref[...]

    return pl.pallas_call(
        body,
        grid=(N // BM,),
        in_specs=[
            pl.BlockSpec((BM, NPH), lambda i: (i, 0)),
            pl.BlockSpec((NPH, D), lambda i: (0, 0)),
            pl.BlockSpec((1, D), lambda i: (0, 0)),
        ],
        out_specs=pl.BlockSpec((BM, D), lambda i: (i, 0)),
        out_shape=jax.ShapeDtypeStruct((N, D), jnp.float32),
    )(adj, sup_a, bias_row)


def _adj_matmul_b(p1, adj, sup_bp):
    def body(p_ref, a_ref, s_ref, o_ref):
        col = lax.broadcasted_iota(jnp.int32, (BM, NPH), 1)
        a = jnp.where(col < NVALID_B, a_ref[...], 0.0)
        o_ref[...] = p_ref[...] + jnp.dot(a, s_ref[...],
                                          preferred_element_type=jnp.float32)

    return pl.pallas_call(
        body,
        grid=(N // BM,),
        in_specs=[
            pl.BlockSpec((BM, D), lambda i: (i, 0)),
            pl.BlockSpec((BM, NPH), lambda i: (i, 1)),
            pl.BlockSpec((NPH, D), lambda i: (0, 0)),
        ],
        out_specs=pl.BlockSpec((BM, D), lambda i: (i, 0)),
        out_shape=jax.ShapeDtypeStruct((N, D), jnp.float32),
    )(p1, adj, sup_bp)


def kernel(input, adj, nbr_idx, weight_0, weights_mask0, bias):
    inp = input.astype(jnp.float32)
    nbr = nbr_idx.astype(jnp.int32)

    nbr_chunks = (jnp.zeros((NP, DEG), jnp.int32).at[:N].set(nbr)
                  .reshape(2, NW, NCHUNK, E))
    wm = weights_mask0.astype(jnp.float32).reshape(2 * D)
    wm_pad = jnp.zeros((D, 8), jnp.float32).at[:, 0].set(wm[:D]).at[:, 1].set(wm[D:])

    scores = _scores_matmul(inp, wm_pad)
    pad8 = jnp.zeros((NP - N, 8), jnp.float32)
    scores_pad = jnp.concatenate([scores, pad8], axis=0)
    asc = scores_pad[:, 0]
    bsc = scores_pad[:, 1]

    w0 = weight_0.astype(jnp.float32)
    adj32 = adj.astype(jnp.float32)
    bias_row = bias.astype(jnp.float32).reshape(1, D)

    agg_a = _sc_aggregate(inp, nbr_chunks[0], asc, bsc, 0)
    agg_b = _sc_aggregate(inp, nbr_chunks[1], asc, bsc, NPH)
    sup_a = _support_matmul(inp[:NPH], agg_a, w0, 1024)
    p1 = _adj_matmul_a(adj32, sup_a, bias_row)
    sup_b = _support_matmul(inp[NPH:N], agg_b[:NVALID_B], w0, 976)
    sup_bp = jnp.concatenate(
        [sup_b, jnp.zeros((NPH - NVALID_B, D), jnp.float32)], axis=0)
    return _adj_matmul_b(p1, adj32, sup_bp)


# speedup vs baseline: 3.0234x; 1.0582x over previous
import functools

import jax
import jax.numpy as jnp
from jax import lax
from jax.experimental import pallas as pl
from jax.experimental.pallas import tpu as pltpu
from jax.experimental.pallas import tpu_sc as plsc

N = 10000
D = 128
DEG = 32

NC = 2
NS = 16
NW = NC * NS
NPW = 320
NP = NW * NPW
C = 4
E = C * DEG
NCHUNK = NPW // C
NBUF = 2
RING = 4
NV = D // 16
TROWS = 624


def _sc_aggregate(inp, nbr_chunks, asc, bsc):
    mesh = plsc.VectorSubcoreMesh(
        core_axis_name="c", subcore_axis_name="s", num_cores=NC, num_subcores=NS)

    @functools.partial(
        pl.kernel,
        out_type=jax.ShapeDtypeStruct((NP, D), jnp.float32),
        mesh=mesh,
        compiler_params=pltpu.CompilerParams(needs_layout_passes=False),
        scratch_types=[
            pltpu.VMEM((NP,), jnp.float32),
            pltpu.VMEM((NPW + 16,), jnp.float32),
            pltpu.VMEM((RING, E), jnp.int32),
            pltpu.VMEM((NBUF, E, D), jnp.float32),
            pltpu.VMEM((RING, C, D), jnp.float32),
            pltpu.VMEM((E,), jnp.float32),
            pltpu.VMEM_SHARED((N, D), jnp.float32),
            pltpu.SemaphoreType.DMA((RING,)),
            pltpu.SemaphoreType.DMA((NBUF,)),
            pltpu.SemaphoreType.DMA((RING,)),
            pltpu.SemaphoreType.DMA,
        ],
    )
    def k(inp_hbm, nbr_hbm, asc_hbm, bsc_hbm, out_hbm,
          bsc_v, asc_v, idx_v, rows_v, acc_v, mask_v, tab_s,
          isem, gsem, osem, tsem):
        sid = lax.axis_index("s")
        wid = sid * NC + lax.axis_index("c")
        nbase = wid * NPW
        tb = sid * TROWS
        pltpu.async_copy(inp_hbm.at[pl.ds(tb, TROWS)],
                         tab_s.at[pl.ds(tb, TROWS)], tsem)

        @pl.when(sid == 0)
        def _():
            pltpu.sync_copy(inp_hbm.at[pl.ds(NS * TROWS, N - NS * TROWS)],
                            tab_s.at[pl.ds(NS * TROWS, N - NS * TROWS)])

        for g in range(RING):
            pltpu.async_copy(nbr_hbm.at[wid, g], idx_v.at[g], isem.at[g])
        pltpu.sync_copy(bsc_hbm, bsc_v)
        pltpu.sync_copy(asc_hbm.at[pl.ds(nbase, NPW)], asc_v.at[pl.ds(0, NPW)])
        pltpu.make_async_copy(inp_hbm.at[pl.ds(tb, TROWS)],
                              tab_s.at[pl.ds(tb, TROWS)], tsem).wait()
        plsc.subcore_barrier()

        for g in range(NBUF):
            pltpu.make_async_copy(nbr_hbm.at[wid, g], idx_v.at[g],
                                  isem.at[g]).wait()
            pltpu.async_copy(tab_s.at[idx_v.at[g]], rows_v.at[g], gsem.at[g])

        @pl.loop(0, NCHUNK, step=RING)
        def _outer(g0):
            for q in range(RING):
                gc = g0 + q
                b = q % NBUF
                pltpu.make_async_copy(
                    tab_s.at[idx_v.at[q]], rows_v.at[b], gsem.at[b]).wait()
                av = asc_v[pl.ds(gc * C, 16)]
                for v in range(E // 16):
                    idx16 = idx_v[q, pl.ds(v * 16, 16)]
                    bs = plsc.load_gather(bsc_v, [idx16])
                    x = bs + lax.broadcast(av[v // 2], (16,))
                    z = jnp.exp(-jnp.abs(x))
                    mask_v[pl.ds(v * 16, 16)] = jnp.where(
                        x >= 0, 1.0 / (1.0 + z), z / (1.0 + z))
                @pl.when(gc + RING < NCHUNK)
                def _():
                    pltpu.async_copy(nbr_hbm.at[wid, gc + RING], idx_v.at[q],
                                     isem.at[q])
                @pl.when(gc >= RING)
                def _():
                    pltpu.make_async_copy(
                        acc_v.at[q],
                        out_hbm.at[pl.ds(nbase + (gc - RING) * C, C)],
                        osem.at[q]).wait()

                @pl.loop(0, C)
                def _node(n):
                    acc = [jnp.zeros((16,), jnp.float32) for _ in range(NV)]
                    for h in range(DEG // 16):
                        mv = mask_v[pl.ds(n * DEG + h * 16, 16)]
                        for j in range(16):
                            e = n * DEG + h * 16 + j
                            m = lax.broadcast(mv[j], (16,))
                            for v in range(NV):
                                acc[v] = acc[v] + m * rows_v[b, e, pl.ds(v * 16, 16)]
                    for v in range(NV):
                        acc_v[q, n, pl.ds(v * 16, 16)] = acc[v]

                pltpu.async_copy(acc_v.at[q],
                                 out_hbm.at[pl.ds(nbase + gc * C, C)], osem.at[q])
                @pl.when(gc + NBUF < NCHUNK)
                def _():
                    q2 = (q + NBUF) % RING
                    pltpu.make_async_copy(nbr_hbm.at[wid, gc + NBUF],
                                          idx_v.at[q2], isem.at[q2]).wait()
                    pltpu.async_copy(tab_s.at[idx_v.at[q2]], rows_v.at[b],
                                     gsem.at[b])

        for q in range(RING):
            pltpu.make_async_copy(
                acc_v.at[q],
                out_hbm.at[pl.ds(nbase + (NCHUNK - RING + q) * C, C)],
                osem.at[q]).wait()

    return k(inp, nbr_chunks, asc, bsc)


def _scores_matmul(inp, wm_pad):
    blk = 1000

    def body(x_ref, w_ref, o_ref):
        o_ref[...] = jnp.dot(x_ref[...], w_ref[...],
                             preferred_element_type=jnp.float32)

    return pl.pallas_call(
        body,
        grid=(N // blk,),
        in_specs=[
            pl.BlockSpec((blk, D), lambda i: (i, 0)),
            pl.BlockSpec((D, 8), lambda i: (0, 0)),
        ],
        out_specs=pl.BlockSpec((blk, 8), lambda i: (i, 0)),
        out_shape=jax.ShapeDtypeStruct((N, 8), jnp.float32),
    )(inp, wm_pad)


def _support_matmul(x, agg, w):
    blk = 1000

    def body(x_ref, g_ref, w_ref, o_ref):
        o_ref[...] = jnp.dot(x_ref[...] + g_ref[...], w_ref[...],
                             preferred_element_type=jnp.float32)

    return pl.pallas_call(
        body,
        grid=(N // blk,),
        in_specs=[
            pl.BlockSpec((blk, D), lambda i: (i, 0)),
            pl.BlockSpec((blk, D), lambda i: (i, 0)),
            pl.BlockSpec((D, D), lambda i: (0, 0)),
        ],
        out_specs=pl.BlockSpec((blk, D), lambda i: (i, 0)),
        out_shape=jax.ShapeDtypeStruct((N, D), jnp.float32),
    )(x, agg, w)


def _adj_matmul(adj, sup, bias_row):
    bm = 400

    def body(a_ref, s_ref, b_ref, o_ref):
        o_ref[...] = jnp.dot(a_ref[...], s_ref[...],
                             preferred_element_type=jnp.float32) + b_ref[...]

    return pl.pallas_call(
        body,
        grid=(N // bm,),
        in_specs=[
            pl.BlockSpec((bm, N), lambda i: (i, 0)),
            pl.BlockSpec((N, D), lambda i: (0, 0)),
            pl.BlockSpec((1, D), lambda i: (0, 0)),
        ],
        out_specs=pl.BlockSpec((bm, D), lambda i: (i, 0)),
        out_shape=jax.ShapeDtypeStruct((N, D), jnp.float32),
    )(adj, sup, bias_row)


def kernel(input, adj, nbr_idx, weight_0, weights_mask0, bias):
    inp = input.astype(jnp.float32)
    nbr = nbr_idx.astype(jnp.int32)

    nbr_chunks = (jnp.zeros((NP, DEG), jnp.int32).at[:N].set(nbr)
                  .reshape(NW, NCHUNK, E))
    wm = weights_mask0.astype(jnp.float32).reshape(2 * D)
    wm_pad = jnp.zeros((D, 8), jnp.float32).at[:, 0].set(wm[:D]).at[:, 1].set(wm[D:])

    scores = _scores_matmul(inp, wm_pad)
    pad8 = jnp.zeros((NP - N, 8), jnp.float32)
    scores_pad = jnp.concatenate([scores, pad8], axis=0)
    asc = scores_pad[:, 0]
    bsc = scores_pad[:, 1]

    agg = _sc_aggregate(inp, nbr_chunks, asc, bsc)
    sup = _support_matmul(inp, agg[:N], weight_0.astype(jnp.float32))
    return _adj_matmul(adj.astype(jnp.float32), sup,
                       bias.astype(jnp.float32).reshape(1, D))
